# baseline (speedup 1.0000x reference)
import jax
import jax.numpy as jnp
from jax import lax
from jax.experimental import pallas as pl
from jax.experimental.pallas import tpu as pltpu
from jax.experimental.pallas import tpu_sc as plsc

_D = 20
_L = 16
_NW = 32
_C = 1024
_NB = 4


def _body(len_hbm, wt_hbm, out_hbm, tab_v, len_v, out_v, *sems):
    n = len_hbm.shape[0]
    per_w = n // _NW
    nc = per_w // _C
    ngrp = nc // _NB
    wid = lax.axis_index("s") * 2 + lax.axis_index("c")
    base = wid * per_w
    sin = sems[:_NB]
    sout = sems[_NB:]

    pltpu.sync_copy(wt_hbm, tab_v)

    for b in range(_NB):
        pltpu.async_copy(
            len_hbm.at[pl.ds(base + b * _C, _C)], len_v.at[b], sin[b])

    def grp(g, _):
        for b in range(_NB):
            ci = g * _NB + b
            row0 = base + ci * _C

            pltpu.make_async_copy(
                len_hbm.at[pl.ds(row0, _C)], len_v.at[b], sin[b]).wait()

            @pl.when(g > 0)
            def _():
                pltpu.make_async_copy(
                    out_v.at[b],
                    out_hbm.at[:, pl.ds(row0 - _NB * _C, _C)],
                    sout[b]).wait()

            @plsc.parallel_loop(0, _C // _L, step=1, unroll=2)
            def emit(gi):
                r0 = gi * _L
                l = len_v[b, pl.ds(r0, _L)]
                e = jnp.minimum(jnp.maximum(l + 4, 0), 7)
                for j in range(_D):
                    v = plsc.load_gather(tab_v, [e + j * 8])
                    out_v[b, j, pl.ds(r0, _L)] = v

            pltpu.async_copy(
                out_v.at[b], out_hbm.at[:, pl.ds(row0, _C)], sout[b])

            @pl.when(g < ngrp - 1)
            def _():
                pltpu.async_copy(
                    len_hbm.at[pl.ds(row0 + _NB * _C, _C)],
                    len_v.at[b], sin[b])
        return 0

    lax.fori_loop(0, ngrp, grp, 0)

    for b in range(_NB):
        pltpu.make_async_copy(
            out_v.at[b],
            out_hbm.at[:, pl.ds(base + (nc - _NB + b) * _C, _C)],
            sout[b]).wait()


def kernel(lengths, W):
    n = lengths.shape[0]
    lengths = lengths.astype(jnp.int32)
    wt = W.astype(jnp.float32).T.reshape(-1)

    mesh = plsc.VectorSubcoreMesh(core_axis_name="c", subcore_axis_name="s")
    out_t = pl.kernel(
        _body,
        out_type=jax.ShapeDtypeStruct((_D, n), jnp.float32),
        mesh=mesh,
        compiler_params=pltpu.CompilerParams(needs_layout_passes=False),
        scratch_types=[
            pltpu.VMEM((_D * 8,), jnp.float32),
            pltpu.VMEM((_NB, _C), jnp.int32),
            pltpu.VMEM((_NB, _D, _C), jnp.float32),
        ] + [pltpu.SemaphoreType.DMA] * (2 * _NB),
    )(lengths, wt)
    return out_t.T


# speedup vs baseline: 1.0075x; 1.0075x over previous
import jax
import jax.numpy as jnp
from jax import lax
from jax.experimental import pallas as pl
from jax.experimental.pallas import tpu as pltpu
from jax.experimental.pallas import tpu_sc as plsc

_D = 20
_L = 16
_NW = 32
_C = 2048


def _body(len_hbm, wt_hbm, out_hbm, tab_v, len_v, out_v,
          si0, si1, so0, so1):
    n = len_hbm.shape[0]
    per_w = n // _NW
    nc = per_w // _C
    npairs = nc // 2
    wid = lax.axis_index("s") * 2 + lax.axis_index("c")
    base = wid * per_w
    sin = (si0, si1)
    sout = (so0, so1)

    pltpu.sync_copy(wt_hbm, tab_v)

    for b in (0, 1):
        pltpu.async_copy(
            len_hbm.at[pl.ds(base + b * _C, _C)], len_v.at[b], sin[b])

    def pair(ci2, _):
        for b in (0, 1):
            ci = ci2 * 2 + b
            row0 = base + ci * _C

            pltpu.make_async_copy(
                len_hbm.at[pl.ds(row0, _C)], len_v.at[b], sin[b]).wait()

            @pl.when(ci2 > 0)
            def _():
                pltpu.make_async_copy(
                    out_v.at[b],
                    out_hbm.at[:, pl.ds(row0 - 2 * _C, _C)],
                    sout[b]).wait()

            @plsc.parallel_loop(0, _C // _L, step=1, unroll=2)
            def emit(gi):
                r0 = gi * _L
                l = len_v[b, pl.ds(r0, _L)]
                e = jnp.minimum(jnp.maximum(l + 4, 0), 7)
                for j in range(_D):
                    v = plsc.load_gather(tab_v, [e + j * 8])
                    out_v[b, j, pl.ds(r0, _L)] = v

            pltpu.async_copy(
                out_v.at[b], out_hbm.at[:, pl.ds(row0, _C)], sout[b])

            @pl.when(ci2 < npairs - 1)
            def _():
                pltpu.async_copy(
                    len_hbm.at[pl.ds(row0 + 2 * _C, _C)],
                    len_v.at[b], sin[b])
        return 0

    lax.fori_loop(0, npairs, pair, 0)

    for b in (0, 1):
        pltpu.make_async_copy(
            out_v.at[b],
            out_hbm.at[:, pl.ds(base + (nc - 2 + b) * _C, _C)],
            sout[b]).wait()


def kernel(lengths, W):
    n = lengths.shape[0]
    lengths = lengths.astype(jnp.int32)
    wt = W.astype(jnp.float32).T.reshape(-1)

    mesh = plsc.VectorSubcoreMesh(core_axis_name="c", subcore_axis_name="s")
    out_t = pl.kernel(
        _body,
        out_type=jax.ShapeDtypeStruct((_D, n), jnp.float32),
        mesh=mesh,
        compiler_params=pltpu.CompilerParams(needs_layout_passes=False),
        scratch_types=[
            pltpu.VMEM((_D * 8,), jnp.float32),
            pltpu.VMEM((2, _C), jnp.int32),
            pltpu.VMEM((2, _D, _C), jnp.float32),
            pltpu.SemaphoreType.DMA,
            pltpu.SemaphoreType.DMA,
            pltpu.SemaphoreType.DMA,
            pltpu.SemaphoreType.DMA,
        ],
    )(lengths, wt)
    return out_t.T
